# baseline (speedup 1.0000x reference)
import functools

import jax
import jax.numpy as jnp
from jax import lax
from jax.experimental import pallas as pl
from jax.experimental.pallas import tpu as pltpu
from jax.experimental.pallas import tpu_sc as plsc

_LANES = 16


def _build(B, S, E, V):
    info = plsc.get_sparse_core_info()
    NC, NS = info.num_cores, info.num_subcores
    NW = NC * NS
    assert B % (NW * 128) == 0 and E == 32
    PC = 8
    n_chunk = S // PC
    assert S % PC == 0 and n_chunk % 2 == 1

    mesh = plsc.VectorSubcoreMesh(core_axis_name="c", subcore_axis_name="s")

    @functools.partial(
        pl.kernel,
        mesh=mesh,
        compiler_params=pltpu.CompilerParams(
            use_tc_tiling_on_sc=False, needs_layout_passes=False),
        out_type=jax.ShapeDtypeStruct((S, E // 8, B // 128, 8, 128), jnp.float32),
        scratch_types=[
            pltpu.VMEM((PC, 128), jnp.int32),
            pltpu.VMEM((PC, 128, E), jnp.float32),
            pltpu.VMEM((PC, 128, E), jnp.float32),
            pltpu.VMEM((PC, E // 8, 8, 128), jnp.float32),
            pltpu.VMEM((S, E), jnp.float32),
            pltpu.SemaphoreType.DMA,
            pltpu.SemaphoreType.DMA,
            pltpu.SemaphoreType.DMA,
        ],
    )
    def k(xt_hbm, tok_hbm, pos_hbm, out_hbm,
          idx_v, grows0, grows1, trows, pos_v,
          sem_g0, sem_g1, sem_o):
        wid = lax.axis_index("s") * NC + lax.axis_index("c")
        col0 = wid * 128
        pltpu.sync_copy(pos_hbm, pos_v)

        iota = lax.iota(jnp.int32, _LANES)
        te_h0 = lax.shift_right_logical(iota, 3)
        te_h1 = te_h0 + 2
        r_idx = lax.bitwise_and(iota, 7)

        def fire(grows_v, sem, c):
            pltpu.sync_copy(
                xt_hbm.at[pl.ds(c * PC, PC), pl.ds(col0, 128)], idx_v)
            for si in range(PC):
                pltpu.async_copy(
                    tok_hbm.at[idx_v.at[si]], grows_v.at[si], sem)

        def drain_gathers(grows_v, sem):
            for si in range(PC):
                pltpu.make_async_copy(
                    tok_hbm.at[idx_v.at[si]], grows_v.at[si], sem).wait()

        def drain_out(sem):
            pltpu.make_async_copy(
                trows, out_hbm.at[pl.ds(0, PC), :, 0], sem).wait()

        def transpose_add(grows_v, c):
            s0 = c * PC
            for si in range(PC):
                si_v = jnp.full((_LANES,), si, jnp.int32)
                p0 = pos_v[s0 + si, pl.ds(0, _LANES)]
                p1 = pos_v[s0 + si, pl.ds(_LANES, _LANES)]

                def body(cc, _):
                    c_v = jnp.full((_LANES,), 0, jnp.int32) + cc
                    v0 = grows_v[si, cc, pl.ds(0, _LANES)] + p0
                    plsc.store_scatter(trows, [si_v, te_h0, r_idx, c_v], v0)
                    v1 = grows_v[si, cc, pl.ds(_LANES, _LANES)] + p1
                    plsc.store_scatter(trows, [si_v, te_h1, r_idx, c_v], v1)
                    return 0

                lax.fori_loop(0, 128, body, 0)

        def put_out(c):
            pltpu.async_copy(
                trows, out_hbm.at[pl.ds(c * PC, PC), :, wid], sem_o)

        def process(grows_P, sem_P, grows_Q, sem_Q, c, prefetch, first):
            drain_gathers(grows_P, sem_P)
            @pl.when(prefetch)
            def _():
                fire(grows_Q, sem_Q, c + 1)
            @pl.when(jnp.logical_not(first))
            def _():
                drain_out(sem_o)
            transpose_add(grows_P, c)
            put_out(c)

        fire(grows0, sem_g0, 0)
        t_ = jnp.bool_(True)
        process(grows0, sem_g0, grows1, sem_g1, 0, t_, t_)

        def pair_body(c2, _):
            cO = 2 * c2 + 1
            process(grows1, sem_g1, grows0, sem_g0, cO, t_, ~t_)
            process(grows0, sem_g0, grows1, sem_g1, cO + 1,
                    c2 < n_chunk // 2 - 1, ~t_)
            return 0

        lax.fori_loop(0, n_chunk // 2, pair_body, 0)
        drain_out(sem_o)

    return k


def kernel(x, tok_table, pos_table):
    B, S = x.shape
    V, E = tok_table.shape
    k = _build(B, S, E, V)
    xt = x.astype(jnp.int32).T
    out5 = k(xt, tok_table, pos_table)
    out = out5.transpose(2, 4, 0, 1, 3).reshape(B, S, E)
    return out


# speedup vs baseline: 1.2540x; 1.2540x over previous
import functools

import jax
import jax.numpy as jnp
from jax import lax
from jax.experimental import pallas as pl
from jax.experimental.pallas import tpu as pltpu
from jax.experimental.pallas import tpu_sc as plsc

_LANES = 16


def _build(B, S, E, V):
    info = plsc.get_sparse_core_info()
    NC, NS = info.num_cores, info.num_subcores
    NW = NC * NS
    assert B % (NW * 128) == 0 and E == 32
    PC = 8
    n_chunk = S // PC
    assert S % PC == 0 and n_chunk % 2 == 1

    mesh = plsc.VectorSubcoreMesh(core_axis_name="c", subcore_axis_name="s")

    @functools.partial(
        pl.kernel,
        mesh=mesh,
        compiler_params=pltpu.CompilerParams(
            use_tc_tiling_on_sc=False, needs_layout_passes=False),
        out_type=jax.ShapeDtypeStruct((S, E // 8, B // 128, 1024), jnp.float32),
        scratch_types=[
            pltpu.VMEM((PC, 128), jnp.int32),
            pltpu.VMEM((PC, 128, E), jnp.float32),
            pltpu.VMEM((PC, 128, E), jnp.float32),
            pltpu.VMEM((PC, E // 8, 1024), jnp.float32),
            pltpu.VMEM((S, E), jnp.float32),
            pltpu.SemaphoreType.DMA,
            pltpu.SemaphoreType.DMA,
            pltpu.SemaphoreType.DMA,
        ],
    )
    def k(xt_hbm, tok_hbm, pos_hbm, out_hbm,
          idx_v, grows0, grows1, trows, pos_v,
          sem_g0, sem_g1, sem_o):
        wid = lax.axis_index("s") * NC + lax.axis_index("c")
        col0 = wid * 128
        pltpu.sync_copy(pos_hbm, pos_v)

        iota = lax.iota(jnp.int32, _LANES)
        te_h0 = lax.shift_right_logical(iota, 3)
        te_h1 = te_h0 + 2
        rk = lax.bitwise_and(iota, 7) * 128

        def fire(grows_v, sem, c):
            pltpu.sync_copy(
                xt_hbm.at[pl.ds(c * PC, PC), pl.ds(col0, 128)], idx_v)
            for si in range(PC):
                pltpu.async_copy(
                    tok_hbm.at[idx_v.at[si]], grows_v.at[si], sem)

        def drain_gathers(grows_v, sem):
            for si in range(PC):
                pltpu.make_async_copy(
                    tok_hbm.at[idx_v.at[si]], grows_v.at[si], sem).wait()

        def drain_out(sem):
            pltpu.make_async_copy(
                trows, out_hbm.at[pl.ds(0, PC), :, 0], sem).wait()

        def transpose_add(grows_v, c):
            s0 = c * PC
            for si in range(PC):
                si_v = jnp.full((_LANES,), si, jnp.int32)
                p0 = pos_v[s0 + si, pl.ds(0, _LANES)]
                p1 = pos_v[s0 + si, pl.ds(_LANES, _LANES)]

                @plsc.parallel_loop(0, 128, unroll=8)
                def _(cc):
                    rc = rk + cc
                    v0 = grows_v[si, cc, pl.ds(0, _LANES)] + p0
                    plsc.store_scatter(trows, [si_v, te_h0, rc], v0)
                    v1 = grows_v[si, cc, pl.ds(_LANES, _LANES)] + p1
                    plsc.store_scatter(trows, [si_v, te_h1, rc], v1)

        def put_out(c):
            pltpu.async_copy(
                trows, out_hbm.at[pl.ds(c * PC, PC), :, wid], sem_o)

        def process(grows_P, sem_P, grows_Q, sem_Q, c, prefetch, first):
            drain_gathers(grows_P, sem_P)
            @pl.when(prefetch)
            def _():
                fire(grows_Q, sem_Q, c + 1)
            @pl.when(jnp.logical_not(first))
            def _():
                drain_out(sem_o)
            transpose_add(grows_P, c)
            put_out(c)

        fire(grows0, sem_g0, 0)
        t_ = jnp.bool_(True)
        process(grows0, sem_g0, grows1, sem_g1, 0, t_, t_)

        def pair_body(c2, _):
            cO = 2 * c2 + 1
            process(grows1, sem_g1, grows0, sem_g0, cO, t_, ~t_)
            process(grows0, sem_g0, grows1, sem_g1, cO + 1,
                    c2 < n_chunk // 2 - 1, ~t_)
            return 0

        lax.fori_loop(0, n_chunk // 2, pair_body, 0)
        drain_out(sem_o)

    return k


def kernel(x, tok_table, pos_table):
    B, S = x.shape
    V, E = tok_table.shape
    k = _build(B, S, E, V)
    xt = x.astype(jnp.int32).T
    out4 = k(xt, tok_table, pos_table)
    out5 = out4.reshape(S, E // 8, B // 128, 8, 128)
    out = out5.transpose(2, 4, 0, 1, 3).reshape(B, S, E)
    return out


# speedup vs baseline: 1.8551x; 1.4794x over previous
import functools

import jax
import jax.numpy as jnp
from jax import lax
from jax.experimental import pallas as pl
from jax.experimental.pallas import tpu as pltpu
from jax.experimental.pallas import tpu_sc as plsc

_LANES = 16


def _build(B, S, E, V):
    info = plsc.get_sparse_core_info()
    NC, NS = info.num_cores, info.num_subcores
    NW = NC * NS
    assert B % (NW * 128) == 0 and E == 32
    PC = 8
    n_chunk = S // PC
    assert S % PC == 0 and n_chunk % 2 == 1

    mesh = plsc.VectorSubcoreMesh(core_axis_name="c", subcore_axis_name="s")

    @functools.partial(
        pl.kernel,
        mesh=mesh,
        compiler_params=pltpu.CompilerParams(
            use_tc_tiling_on_sc=False, needs_layout_passes=False),
        out_type=jax.ShapeDtypeStruct((S, E // 8, B // 128, 8, 128), jnp.float32),
        scratch_types=[
            pltpu.VMEM((PC, 128), jnp.int32),
            pltpu.VMEM((PC, 128, E), jnp.float32),
            pltpu.VMEM((PC, 128, E), jnp.float32),
            pltpu.VMEM((PC, E // 8, 8, 129), jnp.float32),
            pltpu.VMEM((S, E), jnp.float32),
            pltpu.SemaphoreType.DMA,
            pltpu.SemaphoreType.DMA,
            pltpu.SemaphoreType.DMA,
        ],
    )
    def k(xt_hbm, tok_hbm, pos_hbm, out_hbm,
          idx_v, grows0, grows1, trows, pos_v,
          sem_g0, sem_g1, sem_o):
        wid = lax.axis_index("s") * NC + lax.axis_index("c")
        col0 = wid * 128
        pltpu.sync_copy(pos_hbm, pos_v)

        iota = lax.iota(jnp.int32, _LANES)
        te_h0 = lax.shift_right_logical(iota, 3)
        te_h1 = te_h0 + 2
        r_idx = lax.bitwise_and(iota, 7)

        def fire(grows_v, sem, c):
            pltpu.sync_copy(
                xt_hbm.at[pl.ds(c * PC, PC), pl.ds(col0, 128)], idx_v)
            for si in range(PC):
                pltpu.async_copy(
                    tok_hbm.at[idx_v.at[si]], grows_v.at[si], sem)

        def drain_gathers(grows_v, sem):
            for si in range(PC):
                pltpu.make_async_copy(
                    tok_hbm.at[idx_v.at[si]], grows_v.at[si], sem).wait()

        def drain_out(sem):
            pltpu.make_async_copy(
                trows.at[:, :, :, pl.ds(0, 128)],
                out_hbm.at[pl.ds(0, PC), :, 0], sem).wait()

        def transpose_add(grows_v, c):
            s0 = c * PC
            for si in range(PC):
                si_v = jnp.full((_LANES,), si, jnp.int32)
                p0 = pos_v[s0 + si, pl.ds(0, _LANES)]
                p1 = pos_v[s0 + si, pl.ds(_LANES, _LANES)]

                @plsc.parallel_loop(0, 128, unroll=8)
                def _(cc):
                    c_v = jnp.full((_LANES,), 0, jnp.int32) + cc
                    v0 = grows_v[si, cc, pl.ds(0, _LANES)] + p0
                    plsc.store_scatter(trows, [si_v, te_h0, r_idx, c_v], v0)
                    v1 = grows_v[si, cc, pl.ds(_LANES, _LANES)] + p1
                    plsc.store_scatter(trows, [si_v, te_h1, r_idx, c_v], v1)

        def put_out(c):
            pltpu.async_copy(
                trows.at[:, :, :, pl.ds(0, 128)],
                out_hbm.at[pl.ds(c * PC, PC), :, wid], sem_o)

        def process(grows_P, sem_P, grows_Q, sem_Q, c, prefetch, first):
            drain_gathers(grows_P, sem_P)
            @pl.when(prefetch)
            def _():
                fire(grows_Q, sem_Q, c + 1)
            @pl.when(jnp.logical_not(first))
            def _():
                drain_out(sem_o)
            transpose_add(grows_P, c)
            put_out(c)

        fire(grows0, sem_g0, 0)
        t_ = jnp.bool_(True)
        process(grows0, sem_g0, grows1, sem_g1, 0, t_, t_)

        def pair_body(c2, _):
            cO = 2 * c2 + 1
            process(grows1, sem_g1, grows0, sem_g0, cO, t_, ~t_)
            process(grows0, sem_g0, grows1, sem_g1, cO + 1,
                    c2 < n_chunk // 2 - 1, ~t_)
            return 0

        lax.fori_loop(0, n_chunk // 2, pair_body, 0)
        drain_out(sem_o)

    return k


def kernel(x, tok_table, pos_table):
    B, S = x.shape
    V, E = tok_table.shape
    k = _build(B, S, E, V)
    xt = x.astype(jnp.int32).T
    out5 = k(xt, tok_table, pos_table)
    out = out5.transpose(2, 4, 0, 1, 3).reshape(B, S, E)
    return out


# speedup vs baseline: 1.8906x; 1.0191x over previous
import functools

import jax
import jax.numpy as jnp
from jax import lax
from jax.experimental import pallas as pl
from jax.experimental.pallas import tpu as pltpu
from jax.experimental.pallas import tpu_sc as plsc

_LANES = 16


def _build(B, S, E, V):
    info = plsc.get_sparse_core_info()
    NC, NS = info.num_cores, info.num_subcores
    NW = NC * NS
    assert B % (NW * 128) == 0 and E == 32
    PC = 8
    n_chunk = S // PC
    assert S % PC == 0 and n_chunk % 2 == 1

    mesh = plsc.VectorSubcoreMesh(core_axis_name="c", subcore_axis_name="s")

    @functools.partial(
        pl.kernel,
        mesh=mesh,
        compiler_params=pltpu.CompilerParams(
            use_tc_tiling_on_sc=False, needs_layout_passes=False),
        out_type=jax.ShapeDtypeStruct((S, E // 8, B // 128, 8, 128), jnp.float32),
        scratch_types=[
            pltpu.VMEM((PC, 128), jnp.int32),
            pltpu.VMEM((PC, 128, E), jnp.float32),
            pltpu.VMEM((PC, 128, E), jnp.float32),
            pltpu.VMEM((PC, E // 8, 8, 129), jnp.float32),
            pltpu.VMEM((S, E), jnp.float32),
            pltpu.SemaphoreType.DMA,
            pltpu.SemaphoreType.DMA,
            pltpu.SemaphoreType.DMA,
        ],
    )
    def k(xt_hbm, tok_hbm, pos_hbm, out_hbm,
          idx_v, grows0, grows1, trows, pos_v,
          sem_g0, sem_g1, sem_o):
        wid = lax.axis_index("s") * NC + lax.axis_index("c")
        col0 = wid * 128
        pltpu.sync_copy(pos_hbm, pos_v)

        iota = lax.iota(jnp.int32, _LANES)
        te_h0 = lax.shift_right_logical(iota, 3)
        te_h1 = te_h0 + 2
        r_idx = lax.bitwise_and(iota, 7)

        def fire(grows_v, sem, c):
            pltpu.sync_copy(
                xt_hbm.at[pl.ds(c * PC, PC), pl.ds(col0, 128)], idx_v)
            for si in range(PC):
                pltpu.async_copy(
                    tok_hbm.at[idx_v.at[si]], grows_v.at[si], sem)

        def drain_gathers(grows_v, sem):
            for si in range(PC):
                pltpu.make_async_copy(
                    tok_hbm.at[idx_v.at[si]], grows_v.at[si], sem).wait()

        def drain_out(sem):
            pltpu.make_async_copy(
                trows.at[:, :, :, pl.ds(0, 128)],
                out_hbm.at[pl.ds(0, PC), :, 0], sem).wait()

        def transpose_add(grows_v, c):
            s0 = c * PC
            for si in range(PC):
                si_v = jnp.full((_LANES,), si, jnp.int32)
                p0 = pos_v[s0 + si, pl.ds(0, _LANES)]
                p1 = pos_v[s0 + si, pl.ds(_LANES, _LANES)]

                @plsc.parallel_loop(0, 128, unroll=8)
                def _(cc):
                    c_v = jnp.full((_LANES,), 0, jnp.int32) + cc
                    v0 = grows_v[si, cc, pl.ds(0, _LANES)] + p0
                    plsc.store_scatter(trows, [si_v, te_h0, r_idx, c_v], v0)
                    v1 = grows_v[si, cc, pl.ds(_LANES, _LANES)] + p1
                    plsc.store_scatter(trows, [si_v, te_h1, r_idx, c_v], v1)

        def put_out(c):
            pltpu.async_copy(
                trows.at[:, :, :, pl.ds(0, 128)],
                out_hbm.at[pl.ds(c * PC, PC), :, wid], sem_o)

        def process(grows_P, sem_P, grows_Q, sem_Q, c, prefetch, first):
            drain_gathers(grows_P, sem_P)
            @pl.when(prefetch)
            def _():
                fire(grows_Q, sem_Q, c + 1)
            @pl.when(jnp.logical_not(first))
            def _():
                drain_out(sem_o)
            transpose_add(grows_P, c)
            put_out(c)

        fire(grows0, sem_g0, 0)
        t_ = jnp.bool_(True)
        process(grows0, sem_g0, grows1, sem_g1, 0, t_, t_)

        def pair_body(c2, _):
            cO = 2 * c2 + 1
            process(grows1, sem_g1, grows0, sem_g0, cO, t_, ~t_)
            process(grows0, sem_g0, grows1, sem_g1, cO + 1,
                    c2 < n_chunk // 2 - 1, ~t_)
            return 0

        lax.fori_loop(0, n_chunk // 2, pair_body, 0)
        drain_out(sem_o)

    return k


def kernel(x, tok_table, pos_table):
    B, S = x.shape
    V, E = tok_table.shape
    k = _build(B, S, E, V)
    xt = x.astype(jnp.int32).T * 4
    tok_lin = jnp.pad(tok_table, ((0, 0), (0, 3 * E))).reshape(4 * V, E)
    out5 = k(xt, tok_lin, pos_table)
    out = out5.transpose(2, 4, 0, 1, 3).reshape(B, S, E)
    return out


# speedup vs baseline: 1.8937x; 1.0017x over previous
import functools

import jax
import jax.numpy as jnp
from jax import lax
from jax.experimental import pallas as pl
from jax.experimental.pallas import tpu as pltpu
from jax.experimental.pallas import tpu_sc as plsc

_LANES = 16


def _build(B, S, E, V):
    info = plsc.get_sparse_core_info()
    NC, NS = info.num_cores, info.num_subcores
    NW = NC * NS
    assert B % (NW * 128) == 0 and E == 32
    PC = 8
    n_chunk = S // PC
    assert S % PC == 0 and n_chunk % 2 == 1

    mesh = plsc.VectorSubcoreMesh(core_axis_name="c", subcore_axis_name="s")

    @functools.partial(
        pl.kernel,
        mesh=mesh,
        compiler_params=pltpu.CompilerParams(
            use_tc_tiling_on_sc=False, needs_layout_passes=False),
        out_type=jax.ShapeDtypeStruct((S, E // 8, B // 128, 8, 128), jnp.float32),
        scratch_types=[
            pltpu.VMEM((PC, 128), jnp.int32),
            pltpu.VMEM((PC, 128), jnp.int32),
            pltpu.VMEM((PC, 128, E), jnp.float32),
            pltpu.VMEM((PC, 128, E), jnp.float32),
            pltpu.VMEM((PC, E // 8, 8, 129), jnp.float32),
            pltpu.VMEM((S, E), jnp.float32),
            pltpu.SemaphoreType.DMA,
            pltpu.SemaphoreType.DMA,
            pltpu.SemaphoreType.DMA,
        ],
    )
    def k(xt_hbm, tok_hbm, pos_hbm, out_hbm,
          idx0, idx1, grows0, grows1, trows, pos_v,
          sem_g0, sem_g1, sem_o):
        wid = lax.axis_index("s") * NC + lax.axis_index("c")
        col0 = wid * 128
        pltpu.sync_copy(pos_hbm, pos_v)

        iota = lax.iota(jnp.int32, _LANES)
        te_h0 = lax.shift_right_logical(iota, 3)
        te_h1 = te_h0 + 2
        r_idx = lax.bitwise_and(iota, 7)

        def fire(idx_v, grows_v, sem, c):
            pltpu.sync_copy(
                xt_hbm.at[pl.ds(c * PC, PC), pl.ds(col0, 128)], idx_v)
            for si in range(PC):
                pltpu.async_copy(
                    tok_hbm.at[idx_v.at[si]], grows_v.at[si], sem)

        def drain_gathers(idx_v, grows_v, sem):
            for si in range(PC):
                pltpu.make_async_copy(
                    tok_hbm.at[idx_v.at[si]], grows_v.at[si], sem).wait()

        def drain_out(sem):
            pltpu.make_async_copy(
                trows.at[:, :, :, pl.ds(0, 128)],
                out_hbm.at[pl.ds(0, PC), :, 0], sem).wait()

        def transpose_add(grows_v, c):
            s0 = c * PC
            for si in range(PC):
                si_v = jnp.full((_LANES,), si, jnp.int32)
                p0 = pos_v[s0 + si, pl.ds(0, _LANES)]
                p1 = pos_v[s0 + si, pl.ds(_LANES, _LANES)]

                @plsc.parallel_loop(0, 128, unroll=8)
                def _(cc):
                    c_v = jnp.full((_LANES,), 0, jnp.int32) + cc
                    v0 = grows_v[si, cc, pl.ds(0, _LANES)] + p0
                    plsc.store_scatter(trows, [si_v, te_h0, r_idx, c_v], v0)
                    v1 = grows_v[si, cc, pl.ds(_LANES, _LANES)] + p1
                    plsc.store_scatter(trows, [si_v, te_h1, r_idx, c_v], v1)

        def put_out(c):
            pltpu.async_copy(
                trows.at[:, :, :, pl.ds(0, 128)],
                out_hbm.at[pl.ds(c * PC, PC), :, wid], sem_o)

        def process(idx_P, grows_P, sem_P, idx_Q, grows_Q, sem_Q,
                    c, prefetch, first):
            @pl.when(prefetch)
            def _():
                fire(idx_Q, grows_Q, sem_Q, c + 1)
            drain_gathers(idx_P, grows_P, sem_P)
            @pl.when(jnp.logical_not(first))
            def _():
                drain_out(sem_o)
            transpose_add(grows_P, c)
            put_out(c)

        fire(idx0, grows0, sem_g0, 0)
        t_ = jnp.bool_(True)
        process(idx0, grows0, sem_g0, idx1, grows1, sem_g1, 0, t_, t_)

        def pair_body(c2, _):
            cO = 2 * c2 + 1
            process(idx1, grows1, sem_g1, idx0, grows0, sem_g0, cO, t_, ~t_)
            process(idx0, grows0, sem_g0, idx1, grows1, sem_g1, cO + 1,
                    c2 < n_chunk // 2 - 1, ~t_)
            return 0

        lax.fori_loop(0, n_chunk // 2, pair_body, 0)
        drain_out(sem_o)

    return k


def kernel(x, tok_table, pos_table):
    B, S = x.shape
    V, E = tok_table.shape
    k = _build(B, S, E, V)
    xt = x.astype(jnp.int32).T * 4
    tok_lin = jnp.pad(tok_table, ((0, 0), (0, 3 * E))).reshape(4 * V, E)
    out5 = k(xt, tok_lin, pos_table)
    out = out5.transpose(2, 4, 0, 1, 3).reshape(B, S, E)
    return out
